# Initial kernel scaffold; baseline (speedup 1.0000x reference)
import jax
import jax.numpy as jnp
from jax.experimental import pallas as pl


def kernel(tokens, embed_table, bn1_gamma, bn1_beta, k_fwd, r_fwd, b_fwd, k_bwd, r_bwd, b_bwd, bn2_gamma, bn2_beta, W_dense, b_dense):
    raise NotImplementedError("write your pallas kernel here")


# speedup vs baseline: 3.8036x; 3.8036x over previous
import functools

import jax
import jax.numpy as jnp
from jax import lax
from jax.experimental import pallas as pl
from jax.experimental.pallas import tpu as pltpu
from jax.experimental.pallas import tpu_sc as plsc

VOCAB = 100000
EM = 128
U = 256
OUT = 64
B = 1024
T = 200
EPS = 1e-3
N = B * T

_NC = 2
_NS = 16
_NW = _NC * _NS
_CHUNK = 128
_ROWS_PER_W = N // _NW
_CHUNKS_PER_W = _ROWS_PER_W // _CHUNK


def _sc_gather(table, idx):
    mesh = plsc.VectorSubcoreMesh(core_axis_name="c", subcore_axis_name="s")

    @functools.partial(
        pl.kernel,
        mesh=mesh,
        out_type=jax.ShapeDtypeStruct((N, EM), jnp.float32),
        scratch_types=[
            pltpu.VMEM((_CHUNK,), jnp.int32),
            pltpu.VMEM((_CHUNK, EM), jnp.float32),
            pltpu.SemaphoreType.DMA,
        ],
    )
    def k(table_hbm, idx_hbm, x_hbm, idx_v, rows_v, sem):
        wid = lax.axis_index("s") * _NC + lax.axis_index("c")
        w_base = wid * _ROWS_PER_W

        def body(j, carry):
            base = w_base + j * _CHUNK
            pltpu.sync_copy(idx_hbm.at[pl.ds(base, _CHUNK)], idx_v)
            pltpu.async_copy(table_hbm.at[idx_v], rows_v, sem).wait()
            pltpu.sync_copy(rows_v, x_hbm.at[pl.ds(base, _CHUNK)])
            return carry

        lax.fori_loop(0, _CHUNKS_PER_W, body, 0)

    return k(table, idx)


_SROWS = 4096


def _stats_body(x_ref, o_ref):
    i = pl.program_id(0)

    @pl.when(i == 0)
    def _():
        o_ref[...] = jnp.zeros_like(o_ref)

    xb = x_ref[...]
    s = jnp.sum(xb, axis=0, keepdims=True)
    s2 = jnp.sum(xb * xb, axis=0, keepdims=True)
    o_ref[...] += jnp.concatenate([s, s2], axis=0)


def _stats(x2d):
    return pl.pallas_call(
        _stats_body,
        grid=(N // _SROWS,),
        in_specs=[pl.BlockSpec((_SROWS, EM), lambda i: (i, 0))],
        out_specs=pl.BlockSpec((2, EM), lambda i: (0, 0)),
        out_shape=jax.ShapeDtypeStruct((2, EM), jnp.float32),
    )(x2d)


def _rec_body(xf_ref, xb_ref, mf_ref, mb_ref,
              kf_ref, rf_ref, bf_ref, kb_ref, rb_ref, bb_ref,
              g2_ref, be2_ref, wd_ref, bd_ref,
              out_ref,
              hf, cf, hb, cb):
    t = pl.program_id(0)

    @pl.when(t == 0)
    def _():
        hf[...] = jnp.zeros_like(hf)
        cf[...] = jnp.zeros_like(cf)
        hb[...] = jnp.zeros_like(hb)
        cb[...] = jnp.zeros_like(cb)

    def step(x, m, h_ref, c_ref, k_ref, r_ref, b_ref):
        h = h_ref[...]
        c = c_ref[...]
        z = jnp.dot(x, k_ref[...], preferred_element_type=jnp.float32)
        z = z + jnp.dot(h, r_ref[...], preferred_element_type=jnp.float32)
        z = z + b_ref[...]
        gi = jax.nn.sigmoid(z[:, 0 * U:1 * U])
        gf = jax.nn.sigmoid(z[:, 1 * U:2 * U])
        gg = jnp.tanh(z[:, 2 * U:3 * U])
        go = jax.nn.sigmoid(z[:, 3 * U:4 * U])
        c_new = gf * c + gi * gg
        h_new = go * jnp.tanh(c_new)
        h_ref[...] = h + m * (h_new - h)
        c_ref[...] = c + m * (c_new - c)

    step(xf_ref[0], mf_ref[0], hf, cf, kf_ref, rf_ref, bf_ref)
    step(xb_ref[0], mb_ref[0], hb, cb, kb_ref, rb_ref, bb_ref)

    @pl.when(t == T - 1)
    def _():
        hcat = jnp.concatenate([hf[...], hb[...]], axis=1)
        mean2 = jnp.mean(hcat, axis=0, keepdims=True)
        var2 = jnp.mean(hcat * hcat, axis=0, keepdims=True) - mean2 * mean2
        hn = g2_ref[...] * (hcat - mean2) * lax.rsqrt(var2 + EPS) + be2_ref[...]
        lin = jnp.dot(hn, wd_ref[...], preferred_element_type=jnp.float32)
        lin = lin + bd_ref[...]
        mx = jnp.max(lin, axis=1, keepdims=True)
        e = jnp.exp(lin - mx)
        out_ref[...] = e / jnp.sum(e, axis=1, keepdims=True)


def _recurrence(x3, m01, kf, rf, bf2, kb, rb, bb2, g2, be2, wd, bd2):
    specs = [
        pl.BlockSpec((1, B, EM), lambda t: (t, 0, 0)),
        pl.BlockSpec((1, B, EM), lambda t: (T - 1 - t, 0, 0)),
        pl.BlockSpec((1, B, 1), lambda t: (t, 0, 0)),
        pl.BlockSpec((1, B, 1), lambda t: (T - 1 - t, 0, 0)),
        pl.BlockSpec((EM, 4 * U), lambda t: (0, 0)),
        pl.BlockSpec((U, 4 * U), lambda t: (0, 0)),
        pl.BlockSpec((1, 4 * U), lambda t: (0, 0)),
        pl.BlockSpec((EM, 4 * U), lambda t: (0, 0)),
        pl.BlockSpec((U, 4 * U), lambda t: (0, 0)),
        pl.BlockSpec((1, 4 * U), lambda t: (0, 0)),
        pl.BlockSpec((1, 2 * U), lambda t: (0, 0)),
        pl.BlockSpec((1, 2 * U), lambda t: (0, 0)),
        pl.BlockSpec((2 * U, OUT), lambda t: (0, 0)),
        pl.BlockSpec((1, OUT), lambda t: (0, 0)),
    ]
    return pl.pallas_call(
        _rec_body,
        grid=(T,),
        in_specs=specs,
        out_specs=pl.BlockSpec((B, OUT), lambda t: (0, 0)),
        out_shape=jax.ShapeDtypeStruct((B, OUT), jnp.float32),
        scratch_shapes=[pltpu.VMEM((B, U), jnp.float32) for _ in range(4)],
    )(x3, x3, m01, m01, kf, rf, bf2, kb, rb, bb2, g2, be2, wd, bd2)


def kernel(tokens, embed_table, bn1_gamma, bn1_beta, k_fwd, r_fwd, b_fwd,
           k_bwd, r_bwd, b_bwd, bn2_gamma, bn2_beta, W_dense, b_dense):
    tokens = tokens.astype(jnp.int32)
    idx = tokens.T.reshape(N)
    x2d = _sc_gather(embed_table, idx)

    st = _stats(x2d)
    mean1 = st[0] / N
    var1 = st[1] / N - mean1 * mean1
    a1 = bn1_gamma * lax.rsqrt(var1 + EPS)
    c1 = bn1_beta - mean1 * a1
    kf = a1[:, None] * k_fwd
    kb = a1[:, None] * k_bwd
    bf2 = (c1 @ k_fwd + b_fwd)[None]
    bb2 = (c1 @ k_bwd + b_bwd)[None]

    x3 = x2d.reshape(T, B, EM)
    m01 = (tokens != 0).astype(jnp.float32).T.reshape(T, B, 1)
    return _recurrence(x3, m01, kf, r_fwd, bf2, kb, r_bwd, bb2,
                       bn2_gamma[None], bn2_beta[None], W_dense,
                       b_dense[None])
